# baseline (speedup 1.0000x reference)
import functools

import jax
import jax.numpy as jnp
from jax import lax
from jax.experimental import pallas as pl
from jax.experimental.pallas import tpu as pltpu
from jax.experimental.pallas import tpu_sc as plsc

N = 10000
D = 128
H = 8
C = 16
EMB = H * C
E = 320000

NC = 2
NS = 16
NW = NC * NS

CH = 128
KCH = 81
EPT = KCH * CH
E_PAD = NW * EPT
KCHB = 162
EPTB = KCHB * CH
HH = H // NC
NP = 10240
NPS = NP // NS
NPW = NP // NW

BM = 400

_mesh = plsc.VectorSubcoreMesh(core_axis_name="c", subcore_axis_name="s",
                               num_cores=NC, num_subcores=NS)


def _f32(*s):
    return jax.ShapeDtypeStruct(s, jnp.float32)


_GDN = lax.GatherDimensionNumbers(
    offset_dims=(), collapsed_slice_dims=(0,), start_index_map=(0,))


def _splat(vec, lane):
    idx = jnp.full((16, 1), lane, jnp.int32)
    return lax.gather(vec, idx, _GDN, (1,),
                      mode=lax.GatherScatterMode.PROMISE_IN_BOUNDS)


@functools.partial(
    pl.kernel,
    out_type=(_f32(E_PAD, 16), _f32(NC, NP, 16)),
    mesh=_mesh,
    compiler_params=pltpu.CompilerParams(use_tc_tiling_on_sc=False),
    scratch_types=[
        pltpu.VMEM((2, CH), jnp.int32),
        pltpu.VMEM((2, CH), jnp.int32),
        pltpu.VMEM((2, CH, 16), jnp.float32),
        pltpu.VMEM((2, CH, 16), jnp.float32),
        pltpu.VMEM((CH, 16), jnp.float32),
        pltpu.VMEM((NPS, 16), jnp.float32),
        pltpu.VMEM_SHARED((NP, 16), jnp.float32),
        pltpu.SemaphoreType.DMA,
        pltpu.SemaphoreType.DMA,
    ],
)
def _sc_edge_denom(src_hbm, dst_hbm, alsd_hbm, alds_hbm,
                   ex_hbm, den_hbm,
                   sidx, didx, bufs, bufd, exb, zbuf, den_sh, sem0, sem1):
    cid = lax.axis_index("c")
    sid = lax.axis_index("s")
    wid = sid * NC + cid
    sems = (sem0, sem1)

    @pl.loop(0, NPS)
    def _(i):
        zbuf[i] = jnp.zeros((16,), jnp.float32)

    pltpu.sync_copy(zbuf, den_sh.at[pl.ds(sid * NPS, NPS)])
    plsc.subcore_barrier()

    def stage(b, k):
        base = wid * EPT + k * CH
        pltpu.sync_copy(src_hbm.at[pl.ds(base, CH)], sidx.at[b])
        pltpu.sync_copy(dst_hbm.at[pl.ds(base, CH)], didx.at[b])
        pltpu.async_copy(alsd_hbm.at[sidx.at[b]], bufs.at[b], sems[b])
        pltpu.async_copy(alds_hbm.at[didx.at[b]], bufd.at[b], sems[b])

    def compute(b, k):
        pltpu.make_async_copy(alsd_hbm.at[sidx.at[b]], bufs.at[b],
                              sems[b]).wait()
        pltpu.make_async_copy(alds_hbm.at[didx.at[b]], bufd.at[b],
                              sems[b]).wait()

        @pl.loop(0, CH, unroll=8)
        def _(i):
            a = bufs[b, i] + bufd[b, i]
            a = jnp.maximum(a, 0.2 * a)
            exb[i] = jnp.exp(a)

        base = wid * EPT + k * CH
        pltpu.sync_copy(exb, ex_hbm.at[pl.ds(base, CH)])
        pltpu.sync_copy(exb, den_sh.at[didx.at[b]], add=True)

    stage(0, 0)

    @pl.loop(0, KCH // 2)
    def _(t):
        stage(1, 2 * t + 1)
        compute(0, 2 * t)

        @pl.when(t + 1 < KCH // 2)
        def _():
            stage(0, 2 * t + 2)

        compute(1, 2 * t + 1)

    compute_last = KCH % 2
    if compute_last:
        stage(0, KCH - 1)
        compute(0, KCH - 1)

    plsc.subcore_barrier()
    pltpu.sync_copy(den_sh.at[pl.ds(sid * NPS, NPS)],
                    den_hbm.at[cid, pl.ds(sid * NPS, NPS)])


@functools.partial(
    pl.kernel,
    out_type=_f32(NC, NP, HH, 16),
    mesh=_mesh,
    compiler_params=pltpu.CompilerParams(use_tc_tiling_on_sc=False),
    scratch_types=[
        pltpu.VMEM((2, CH), jnp.int32),
        pltpu.VMEM((2, CH), jnp.int32),
        pltpu.VMEM((2, CH, HH, 16), jnp.float32),
        pltpu.VMEM((2, CH, HH, 16), jnp.float32),
        pltpu.VMEM((2, CH, 16), jnp.float32),
        pltpu.VMEM((NPS, HH, 16), jnp.float32),
        pltpu.VMEM_SHARED((NP, HH, 16), jnp.float32),
        pltpu.SemaphoreType.DMA,
        pltpu.SemaphoreType.DMA,
        pltpu.SemaphoreType.DMA,
        pltpu.SemaphoreType.DMA,
    ],
)
def _sc_message_pass(src_hbm, dst_hbm, ex_hbm, xhh_hbm, out_hbm,
                     sidx, didx, rows, msg, exb, zb, out_sh,
                     gsem0, gsem1, ssem0, ssem1):
    cid = lax.axis_index("c")
    sid = lax.axis_index("s")
    z16 = jnp.zeros((16,), jnp.float32)
    gsems = (gsem0, gsem1)
    ssems = (ssem0, ssem1)

    @pl.loop(0, NPS)
    def _(i):
        for h in range(HH):
            zb[i, h] = z16

    pltpu.sync_copy(zb, out_sh.at[pl.ds(sid * NPS, NPS)])
    plsc.subcore_barrier()

    def wait_scatter(b):
        pltpu.make_async_copy(msg.at[b], out_sh.at[didx.at[b]],
                              ssems[b]).wait()

    def stage(b, k):
        base = sid * EPTB + k * CH
        pltpu.sync_copy(src_hbm.at[pl.ds(base, CH)], sidx.at[b])
        pltpu.sync_copy(dst_hbm.at[pl.ds(base, CH)], didx.at[b])
        pltpu.sync_copy(ex_hbm.at[pl.ds(base, CH)], exb.at[b])
        pltpu.async_copy(xhh_hbm.at[cid].at[sidx.at[b]],
                         rows.at[b], gsems[b])

    def compute(b):
        pltpu.make_async_copy(xhh_hbm.at[cid].at[sidx.at[b]],
                              rows.at[b], gsems[b]).wait()
        hb = cid * HH

        @pl.loop(0, CH, unroll=8)
        def _(i):
            er = exb[b, i]
            for h in range(HH):
                msg[b, i, h] = rows[b, i, h] * _splat(er, hb + h)

        pltpu.async_copy(msg.at[b], out_sh.at[didx.at[b]], ssems[b],
                         add=True)

    stage(0, 0)

    @pl.loop(0, KCHB // 2)
    def _(t):
        @pl.when(t > 0)
        def _():
            wait_scatter(1)

        stage(1, 2 * t + 1)
        compute(0)

        @pl.when(t + 1 < KCHB // 2)
        def _():
            wait_scatter(0)
            stage(0, 2 * t + 2)

        compute(1)

    wait_scatter(0)
    wait_scatter(1)
    plsc.subcore_barrier()
    pltpu.sync_copy(out_sh.at[pl.ds(sid * NPS, NPS)],
                    out_hbm.at[cid, pl.ds(sid * NPS, NPS)])


def _tc_first_body(x_ref, w0_ref, w1_ref, wa_ref, h_ref, xh_ref, al_ref):
    h = jnp.dot(x_ref[...], w0_ref[...], preferred_element_type=jnp.float32)
    h_ref[...] = h
    xh_ref[...] = jnp.dot(h, w1_ref[...], preferred_element_type=jnp.float32)
    al_ref[...] = jnp.dot(h, wa_ref[...], preferred_element_type=jnp.float32)


def _tc_first(x, w0, w1, wa):
    return pl.pallas_call(
        _tc_first_body,
        grid=(N // BM,),
        in_specs=[pl.BlockSpec((BM, D), lambda i: (i, 0)),
                  pl.BlockSpec((D, EMB), lambda i: (0, 0)),
                  pl.BlockSpec((EMB, EMB), lambda i: (0, 0)),
                  pl.BlockSpec((EMB, EMB), lambda i: (0, 0))],
        out_specs=[pl.BlockSpec((BM, EMB), lambda i: (i, 0))] * 3,
        out_shape=[_f32(N, EMB)] * 3,
    )(x, w0, w1, wa)


def _tc_post_body(h_ref, a_ref, d0_ref, d1_ref, b_ref, u_ref, st_ref):
    jrow = lax.broadcasted_iota(jnp.int32, (16, EMB), 0)
    ccol = lax.broadcasted_iota(jnp.int32, (16, EMB), 1) // C
    expm = jnp.where(jrow == ccol, 1.0, 0.0).astype(jnp.float32)
    den16 = d0_ref[0] + d1_ref[0]
    den128 = jnp.dot(den16, expm, preferred_element_type=jnp.float32)
    u = (h_ref[...]
         + a_ref[...] / (den128 + 1e-16)
         + b_ref[...])
    u_ref[...] = u

    @pl.when(pl.program_id(0) == 0)
    def _():
        st_ref[...] = jnp.zeros_like(st_ref)

    s0 = jnp.sum(u, axis=0, keepdims=True)
    s1 = jnp.sum(u * u, axis=0, keepdims=True)
    st_ref[...] += jnp.concatenate(
        [s0, s1, jnp.zeros((6, EMB), jnp.float32)], axis=0)


def _tc_post(h, agg, den, bias):
    return pl.pallas_call(
        _tc_post_body,
        grid=(N // BM,),
        in_specs=[pl.BlockSpec((BM, EMB), lambda i: (i, 0)),
                  pl.BlockSpec((BM, EMB), lambda i: (i, 0)),
                  pl.BlockSpec((1, BM, 16), lambda i: (0, i, 0)),
                  pl.BlockSpec((1, BM, 16), lambda i: (1, i, 0)),
                  pl.BlockSpec((1, EMB), lambda i: (0, 0))],
        out_specs=[pl.BlockSpec((BM, EMB), lambda i: (i, 0)),
                   pl.BlockSpec((8, EMB), lambda i: (0, 0))],
        out_shape=[_f32(N, EMB), _f32(8, EMB)],
    )(h, agg, den, den, bias)


def _tc_next_body(u_ref, sc_ref, sh_ref, w_ref, wa_ref, h_ref, xh_ref, al_ref):
    hcur = u_ref[...] * sc_ref[...] + sh_ref[...]
    h_ref[...] = hcur
    xh_ref[...] = jnp.dot(hcur, w_ref[...], preferred_element_type=jnp.float32)
    al_ref[...] = jnp.dot(hcur, wa_ref[...], preferred_element_type=jnp.float32)


def _tc_next(u, scale, shift, w, wa):
    return pl.pallas_call(
        _tc_next_body,
        grid=(N // BM,),
        in_specs=[pl.BlockSpec((BM, EMB), lambda i: (i, 0)),
                  pl.BlockSpec((1, EMB), lambda i: (0, 0)),
                  pl.BlockSpec((1, EMB), lambda i: (0, 0)),
                  pl.BlockSpec((EMB, EMB), lambda i: (0, 0)),
                  pl.BlockSpec((EMB, EMB), lambda i: (0, 0))],
        out_specs=[pl.BlockSpec((BM, EMB), lambda i: (i, 0))] * 3,
        out_shape=[_f32(N, EMB)] * 3,
    )(u, scale, shift, w, wa)


def _tc_final_body(u_ref, sc_ref, sh_ref, h_ref):
    h_ref[...] = u_ref[...] * sc_ref[...] + sh_ref[...]


def _tc_final(u, scale, shift):
    return pl.pallas_call(
        _tc_final_body,
        grid=(N // BM,),
        in_specs=[pl.BlockSpec((BM, EMB), lambda i: (i, 0)),
                  pl.BlockSpec((1, EMB), lambda i: (0, 0)),
                  pl.BlockSpec((1, EMB), lambda i: (0, 0))],
        out_specs=pl.BlockSpec((BM, EMB), lambda i: (i, 0)),
        out_shape=_f32(N, EMB),
    )(u, scale, shift)


def _prep_edges(edge_index):
    src0 = edge_index[0]
    dst0 = edge_index[1]
    loops = jnp.arange(N, dtype=jnp.int32)
    keep = src0 != dst0
    padn = E_PAD - (E + N)
    src = jnp.concatenate([src0, loops, jnp.zeros((padn,), jnp.int32)])
    dst = jnp.concatenate([jnp.where(keep, dst0, N), loops,
                           jnp.full((padn,), N, jnp.int32)])
    return src, dst


def _attn_w(w, a_s, a_d):
    wr = w.reshape(EMB, H, C)
    ws = (wr * a_s[None]).sum(-1)
    wd = (wr * a_d[None]).sum(-1)
    wa = jnp.concatenate([ws, wd], axis=1)
    return jnp.pad(wa, ((0, 0), (0, EMB - 2 * H)))


def _al_tables(al):
    alsd = al[:, :16]
    alds = jnp.concatenate([al[:, 8:16], al[:, :8]], axis=1)
    pad = ((0, NP - N), (0, 0))
    return jnp.pad(alsd, pad), jnp.pad(alds, pad)


def _bn_coeffs(st, g, bt):
    mean = st[0] / N
    var = st[1] / N - mean * mean
    s = g * lax.rsqrt(var + 1e-5)
    return s.reshape(1, EMB), (bt - mean * s).reshape(1, EMB)


def _gat_residual(h, xh, al, src, dst, bias):
    alsd, alds = _al_tables(al)
    ex, den = _sc_edge_denom(src, dst, alsd, alds)
    xhh = xh.reshape(N, NC, HH, 16).transpose(1, 0, 2, 3)
    outp = _sc_message_pass(src, dst, ex, xhh)
    agg = jnp.concatenate([outp[0, :N], outp[1, :N]], axis=1).reshape(N, EMB)
    return _tc_post(h, agg, den, bias.reshape(1, EMB))


def kernel(x, edge_index, W0, W1, as1, ad1, b1, g1, bt1,
           W2, as2, ad2, b2, g2, bt2, W3, as3, ad3, b3, g3, bt3):
    src, dst = _prep_edges(edge_index)
    h, xh, al = _tc_first(x, W0, W1, _attn_w(W1, as1, ad1))
    u, st = _gat_residual(h, xh, al, src, dst, b1)
    sc_, sh_ = _bn_coeffs(st, g1, bt1)
    h, xh, al = _tc_next(u, sc_, sh_, W2, _attn_w(W2, as2, ad2))
    u, st = _gat_residual(h, xh, al, src, dst, b2)
    sc_, sh_ = _bn_coeffs(st, g2, bt2)
    h, xh, al = _tc_next(u, sc_, sh_, W3, _attn_w(W3, as3, ad3))
    u, st = _gat_residual(h, xh, al, src, dst, b3)
    sc_, sh_ = _bn_coeffs(st, g3, bt3)
    return _tc_final(u, sc_, sh_)


# speedup vs baseline: 1.4074x; 1.4074x over previous
import functools

import jax
import jax.numpy as jnp
from jax import lax
from jax.experimental import pallas as pl
from jax.experimental.pallas import tpu as pltpu
from jax.experimental.pallas import tpu_sc as plsc

N = 10000
D = 128
H = 8
C = 16
EMB = H * C
E = 320000

NC = 2
NS = 16
NW = NC * NS

CH = 128
KCH = 81
EPT = KCH * CH
E_PAD = NW * EPT
KCHB = 162
EPTB = KCHB * CH
HH = H // NC
NP = 10240
NPS = NP // NS
NPW = NP // NW

BM = 400

_mesh = plsc.VectorSubcoreMesh(core_axis_name="c", subcore_axis_name="s",
                               num_cores=NC, num_subcores=NS)


def _f32(*s):
    return jax.ShapeDtypeStruct(s, jnp.float32)


_GDN = lax.GatherDimensionNumbers(
    offset_dims=(), collapsed_slice_dims=(0,), start_index_map=(0,))


def _splat(vec, lane):
    idx = jnp.full((16, 1), lane, jnp.int32)
    return lax.gather(vec, idx, _GDN, (1,),
                      mode=lax.GatherScatterMode.PROMISE_IN_BOUNDS)


@functools.partial(
    pl.kernel,
    out_type=(_f32(E_PAD, 16), _f32(NC, NP, 16)),
    mesh=_mesh,
    compiler_params=pltpu.CompilerParams(use_tc_tiling_on_sc=False),
    scratch_types=[
        pltpu.VMEM((2, CH), jnp.int32),
        pltpu.VMEM((2, CH), jnp.int32),
        pltpu.VMEM((2, CH, 16), jnp.float32),
        pltpu.VMEM((2, CH, 16), jnp.float32),
        pltpu.VMEM((CH, 16), jnp.float32),
        pltpu.VMEM((NPS, 16), jnp.float32),
        pltpu.VMEM_SHARED((NP, 16), jnp.float32),
        pltpu.SemaphoreType.DMA,
        pltpu.SemaphoreType.DMA,
    ],
)
def _sc_edge_denom(src_hbm, dst_hbm, alsd_hbm, alds_hbm,
                   ex_hbm, den_hbm,
                   sidx, didx, bufs, bufd, exb, zbuf, den_sh, sem0, sem1):
    cid = lax.axis_index("c")
    sid = lax.axis_index("s")
    wid = sid * NC + cid
    sems = (sem0, sem1)

    @pl.loop(0, NPS)
    def _(i):
        zbuf[i] = jnp.zeros((16,), jnp.float32)

    pltpu.sync_copy(zbuf, den_sh.at[pl.ds(sid * NPS, NPS)])
    plsc.subcore_barrier()

    def stage(b, k):
        base = wid * EPT + k * CH
        pltpu.sync_copy(src_hbm.at[pl.ds(base, CH)], sidx.at[b])
        pltpu.sync_copy(dst_hbm.at[pl.ds(base, CH)], didx.at[b])
        pltpu.async_copy(alsd_hbm.at[sidx.at[b]], bufs.at[b], sems[b])
        pltpu.async_copy(alds_hbm.at[didx.at[b]], bufd.at[b], sems[b])

    def compute(b, k):
        pltpu.make_async_copy(alsd_hbm.at[sidx.at[b]], bufs.at[b],
                              sems[b]).wait()
        pltpu.make_async_copy(alds_hbm.at[didx.at[b]], bufd.at[b],
                              sems[b]).wait()

        @pl.loop(0, CH)
        def _(i):
            a = bufs[b, i] + bufd[b, i]
            a = jnp.maximum(a, 0.2 * a)
            exb[i] = jnp.exp(a)

        base = wid * EPT + k * CH
        pltpu.sync_copy(exb, ex_hbm.at[pl.ds(base, CH)])
        pltpu.sync_copy(exb, den_sh.at[didx.at[b]], add=True)

    stage(0, 0)

    @pl.loop(0, KCH // 2)
    def _(t):
        stage(1, 2 * t + 1)
        compute(0, 2 * t)

        @pl.when(t + 1 < KCH // 2)
        def _():
            stage(0, 2 * t + 2)

        compute(1, 2 * t + 1)

    compute_last = KCH % 2
    if compute_last:
        stage(0, KCH - 1)
        compute(0, KCH - 1)

    plsc.subcore_barrier()
    pltpu.sync_copy(den_sh.at[pl.ds(sid * NPS, NPS)],
                    den_hbm.at[cid, pl.ds(sid * NPS, NPS)])


@functools.partial(
    pl.kernel,
    out_type=_f32(NC, NP, HH, 16),
    mesh=_mesh,
    compiler_params=pltpu.CompilerParams(use_tc_tiling_on_sc=False),
    scratch_types=[
        pltpu.VMEM((2, CH), jnp.int32),
        pltpu.VMEM((2, CH), jnp.int32),
        pltpu.VMEM((2, CH, HH, 16), jnp.float32),
        pltpu.VMEM((2, CH, HH, 16), jnp.float32),
        pltpu.VMEM((2, CH, 16), jnp.float32),
        pltpu.VMEM((NPS, HH, 16), jnp.float32),
        pltpu.VMEM_SHARED((NP, HH, 16), jnp.float32),
        pltpu.SemaphoreType.DMA,
        pltpu.SemaphoreType.DMA,
        pltpu.SemaphoreType.DMA,
        pltpu.SemaphoreType.DMA,
    ],
)
def _sc_message_pass(src_hbm, dst_hbm, ex_hbm, xhh_hbm, out_hbm,
                     sidx, didx, rows, msg, exb, zb, out_sh,
                     gsem0, gsem1, ssem0, ssem1):
    cid = lax.axis_index("c")
    sid = lax.axis_index("s")
    z16 = jnp.zeros((16,), jnp.float32)
    gsems = (gsem0, gsem1)
    ssems = (ssem0, ssem1)

    @pl.loop(0, NPS)
    def _(i):
        for h in range(HH):
            zb[i, h] = z16

    pltpu.sync_copy(zb, out_sh.at[pl.ds(sid * NPS, NPS)])
    plsc.subcore_barrier()

    def wait_scatter(b):
        pltpu.make_async_copy(msg.at[b], out_sh.at[didx.at[b]],
                              ssems[b]).wait()

    def stage(b, k):
        base = sid * EPTB + k * CH
        pltpu.sync_copy(src_hbm.at[pl.ds(base, CH)], sidx.at[b])
        pltpu.sync_copy(dst_hbm.at[pl.ds(base, CH)], didx.at[b])
        pltpu.sync_copy(ex_hbm.at[pl.ds(base, CH)], exb.at[b])
        pltpu.async_copy(xhh_hbm.at[cid].at[sidx.at[b]],
                         rows.at[b], gsems[b])

    def compute(b):
        pltpu.make_async_copy(xhh_hbm.at[cid].at[sidx.at[b]],
                              rows.at[b], gsems[b]).wait()
        hb = cid * HH

        @pl.loop(0, CH)
        def _(i):
            er = exb[b, i]
            for h in range(HH):
                msg[b, i, h] = rows[b, i, h] * _splat(er, hb + h)

        pltpu.async_copy(msg.at[b], out_sh.at[didx.at[b]], ssems[b],
                         add=True)

    stage(0, 0)

    @pl.loop(0, KCHB // 2)
    def _(t):
        @pl.when(t > 0)
        def _():
            wait_scatter(1)

        stage(1, 2 * t + 1)
        compute(0)

        @pl.when(t + 1 < KCHB // 2)
        def _():
            wait_scatter(0)
            stage(0, 2 * t + 2)

        compute(1)

    wait_scatter(0)
    wait_scatter(1)
    plsc.subcore_barrier()
    pltpu.sync_copy(out_sh.at[pl.ds(sid * NPS, NPS)],
                    out_hbm.at[cid, pl.ds(sid * NPS, NPS)])


def _tc_first_body(x_ref, w0_ref, w1_ref, wa_ref, h_ref, xh_ref, al_ref):
    h = jnp.dot(x_ref[...], w0_ref[...], preferred_element_type=jnp.float32)
    h_ref[...] = h
    xh_ref[...] = jnp.dot(h, w1_ref[...], preferred_element_type=jnp.float32)
    al_ref[...] = jnp.dot(h, wa_ref[...], preferred_element_type=jnp.float32)


def _tc_first(x, w0, w1, wa):
    return pl.pallas_call(
        _tc_first_body,
        grid=(N // BM,),
        in_specs=[pl.BlockSpec((BM, D), lambda i: (i, 0)),
                  pl.BlockSpec((D, EMB), lambda i: (0, 0)),
                  pl.BlockSpec((EMB, EMB), lambda i: (0, 0)),
                  pl.BlockSpec((EMB, EMB), lambda i: (0, 0))],
        out_specs=[pl.BlockSpec((BM, EMB), lambda i: (i, 0))] * 3,
        out_shape=[_f32(N, EMB)] * 3,
    )(x, w0, w1, wa)


def _tc_post_body(h_ref, a_ref, d0_ref, d1_ref, b_ref, u_ref, st_ref):
    jrow = lax.broadcasted_iota(jnp.int32, (16, EMB), 0)
    ccol = lax.broadcasted_iota(jnp.int32, (16, EMB), 1) // C
    expm = jnp.where(jrow == ccol, 1.0, 0.0).astype(jnp.float32)
    den16 = d0_ref[0] + d1_ref[0]
    den128 = jnp.dot(den16, expm, preferred_element_type=jnp.float32)
    u = (h_ref[...]
         + a_ref[...] / (den128 + 1e-16)
         + b_ref[...])
    u_ref[...] = u

    @pl.when(pl.program_id(0) == 0)
    def _():
        st_ref[...] = jnp.zeros_like(st_ref)

    s0 = jnp.sum(u, axis=0, keepdims=True)
    s1 = jnp.sum(u * u, axis=0, keepdims=True)
    st_ref[...] += jnp.concatenate(
        [s0, s1, jnp.zeros((6, EMB), jnp.float32)], axis=0)


def _tc_post(h, agg, den, bias):
    return pl.pallas_call(
        _tc_post_body,
        grid=(N // BM,),
        in_specs=[pl.BlockSpec((BM, EMB), lambda i: (i, 0)),
                  pl.BlockSpec((BM, EMB), lambda i: (i, 0)),
                  pl.BlockSpec((1, BM, 16), lambda i: (0, i, 0)),
                  pl.BlockSpec((1, BM, 16), lambda i: (1, i, 0)),
                  pl.BlockSpec((1, EMB), lambda i: (0, 0))],
        out_specs=[pl.BlockSpec((BM, EMB), lambda i: (i, 0)),
                   pl.BlockSpec((8, EMB), lambda i: (0, 0))],
        out_shape=[_f32(N, EMB), _f32(8, EMB)],
    )(h, agg, den, den, bias)


def _tc_next_body(u_ref, sc_ref, sh_ref, w_ref, wa_ref, h_ref, xh_ref, al_ref):
    hcur = u_ref[...] * sc_ref[...] + sh_ref[...]
    h_ref[...] = hcur
    xh_ref[...] = jnp.dot(hcur, w_ref[...], preferred_element_type=jnp.float32)
    al_ref[...] = jnp.dot(hcur, wa_ref[...], preferred_element_type=jnp.float32)


def _tc_next(u, scale, shift, w, wa):
    return pl.pallas_call(
        _tc_next_body,
        grid=(N // BM,),
        in_specs=[pl.BlockSpec((BM, EMB), lambda i: (i, 0)),
                  pl.BlockSpec((1, EMB), lambda i: (0, 0)),
                  pl.BlockSpec((1, EMB), lambda i: (0, 0)),
                  pl.BlockSpec((EMB, EMB), lambda i: (0, 0)),
                  pl.BlockSpec((EMB, EMB), lambda i: (0, 0))],
        out_specs=[pl.BlockSpec((BM, EMB), lambda i: (i, 0))] * 3,
        out_shape=[_f32(N, EMB)] * 3,
    )(u, scale, shift, w, wa)


def _tc_final_body(u_ref, sc_ref, sh_ref, h_ref):
    h_ref[...] = u_ref[...] * sc_ref[...] + sh_ref[...]


def _tc_final(u, scale, shift):
    return pl.pallas_call(
        _tc_final_body,
        grid=(N // BM,),
        in_specs=[pl.BlockSpec((BM, EMB), lambda i: (i, 0)),
                  pl.BlockSpec((1, EMB), lambda i: (0, 0)),
                  pl.BlockSpec((1, EMB), lambda i: (0, 0))],
        out_specs=pl.BlockSpec((BM, EMB), lambda i: (i, 0)),
        out_shape=_f32(N, EMB),
    )(u, scale, shift)


def _prep_edges(edge_index):
    src0 = edge_index[0]
    dst0 = edge_index[1]
    loops = jnp.arange(N, dtype=jnp.int32)
    keep = src0 != dst0
    padn = E_PAD - (E + N)
    src = jnp.concatenate([src0, loops, jnp.zeros((padn,), jnp.int32)])
    dst = jnp.concatenate([jnp.where(keep, dst0, N), loops,
                           jnp.full((padn,), N, jnp.int32)])
    return src, dst


def _attn_w(w, a_s, a_d):
    wr = w.reshape(EMB, H, C)
    ws = (wr * a_s[None]).sum(-1)
    wd = (wr * a_d[None]).sum(-1)
    wa = jnp.concatenate([ws, wd], axis=1)
    return jnp.pad(wa, ((0, 0), (0, EMB - 2 * H)))


def _al_tables(al):
    alsd = al[:, :16]
    alds = jnp.concatenate([al[:, 8:16], al[:, :8]], axis=1)
    pad = ((0, NP - N), (0, 0))
    return jnp.pad(alsd, pad), jnp.pad(alds, pad)


def _bn_coeffs(st, g, bt):
    mean = st[0] / N
    var = st[1] / N - mean * mean
    s = g * lax.rsqrt(var + 1e-5)
    return s.reshape(1, EMB), (bt - mean * s).reshape(1, EMB)


def _gat_residual(h, xh, al, src, dst, bias):
    alsd, alds = _al_tables(al)
    ex, den = _sc_edge_denom(src, dst, alsd, alds)
    xhh = xh.reshape(N, NC, HH, 16).transpose(1, 0, 2, 3)
    outp = _sc_message_pass(src, dst, ex, xhh)
    agg = jnp.concatenate([outp[0, :N], outp[1, :N]], axis=1).reshape(N, EMB)
    return _tc_post(h, agg, den, bias.reshape(1, EMB))


def kernel(x, edge_index, W0, W1, as1, ad1, b1, g1, bt1,
           W2, as2, ad2, b2, g2, bt2, W3, as3, ad3, b3, g3, bt3):
    src, dst = _prep_edges(edge_index)
    h, xh, al = _tc_first(x, W0, W1, _attn_w(W1, as1, ad1))
    u, st = _gat_residual(h, xh, al, src, dst, b1)
    sc_, sh_ = _bn_coeffs(st, g1, bt1)
    h, xh, al = _tc_next(u, sc_, sh_, W2, _attn_w(W2, as2, ad2))
    u, st = _gat_residual(h, xh, al, src, dst, b2)
    sc_, sh_ = _bn_coeffs(st, g2, bt2)
    h, xh, al = _tc_next(u, sc_, sh_, W3, _attn_w(W3, as3, ad3))
    u, st = _gat_residual(h, xh, al, src, dst, b3)
    sc_, sh_ = _bn_coeffs(st, g3, bt3)
    return _tc_final(u, sc_, sh_)


# speedup vs baseline: 1.5792x; 1.1220x over previous
import functools

import jax
import jax.numpy as jnp
from jax import lax
from jax.experimental import pallas as pl
from jax.experimental.pallas import tpu as pltpu
from jax.experimental.pallas import tpu_sc as plsc

N = 10000
D = 128
H = 8
C = 16
EMB = H * C
E = 320000

NC = 2
NS = 16
NW = NC * NS

CH = 128
KCH = 81
EPT = KCH * CH
E_PAD = NW * EPT
KCHB = 162
EPTB = KCHB * CH
HH = H // NC
NP = 10240
NPS = NP // NS
NPW = NP // NW

BM = 400

_mesh = plsc.VectorSubcoreMesh(core_axis_name="c", subcore_axis_name="s",
                               num_cores=NC, num_subcores=NS)


def _f32(*s):
    return jax.ShapeDtypeStruct(s, jnp.float32)


_GDN = lax.GatherDimensionNumbers(
    offset_dims=(), collapsed_slice_dims=(0,), start_index_map=(0,))


def _splat(vec, lane):
    idx = jnp.full((16, 1), lane, jnp.int32)
    return lax.gather(vec, idx, _GDN, (1,),
                      mode=lax.GatherScatterMode.PROMISE_IN_BOUNDS)


@functools.partial(
    pl.kernel,
    out_type=(_f32(E_PAD, 16), _f32(NC, NP, 16)),
    mesh=_mesh,
    compiler_params=pltpu.CompilerParams(use_tc_tiling_on_sc=False),
    scratch_types=[
        pltpu.VMEM((2, CH), jnp.int32),
        pltpu.VMEM((2, CH), jnp.int32),
        pltpu.VMEM((2, CH, 16), jnp.float32),
        pltpu.VMEM((2, CH, 16), jnp.float32),
        pltpu.VMEM((CH, 16), jnp.float32),
        pltpu.VMEM((NPS, 16), jnp.float32),
        pltpu.VMEM_SHARED((NP, 16), jnp.float32),
        pltpu.SemaphoreType.DMA,
        pltpu.SemaphoreType.DMA,
    ],
)
def _sc_edge_denom(src_hbm, dst_hbm, alsd_hbm, alds_hbm,
                   ex_hbm, den_hbm,
                   sidx, didx, bufs, bufd, exb, zbuf, den_sh, sem0, sem1):
    cid = lax.axis_index("c")
    sid = lax.axis_index("s")
    wid = sid * NC + cid
    sems = (sem0, sem1)

    @pl.loop(0, NPS)
    def _(i):
        zbuf[i] = jnp.zeros((16,), jnp.float32)

    pltpu.sync_copy(zbuf, den_sh.at[pl.ds(sid * NPS, NPS)])
    plsc.subcore_barrier()

    def stage(b, k):
        base = wid * EPT + k * CH
        pltpu.sync_copy(src_hbm.at[pl.ds(base, CH)], sidx.at[b])
        pltpu.sync_copy(dst_hbm.at[pl.ds(base, CH)], didx.at[b])
        pltpu.async_copy(alsd_hbm.at[sidx.at[b]], bufs.at[b], sems[b])
        pltpu.async_copy(alds_hbm.at[didx.at[b]], bufd.at[b], sems[b])

    def compute(b, k):
        pltpu.make_async_copy(alsd_hbm.at[sidx.at[b]], bufs.at[b],
                              sems[b]).wait()
        pltpu.make_async_copy(alds_hbm.at[didx.at[b]], bufd.at[b],
                              sems[b]).wait()

        @pl.loop(0, CH)
        def _(i):
            a = bufs[b, i] + bufd[b, i]
            a = jnp.maximum(a, 0.2 * a)
            exb[i] = jnp.exp(a)

        base = wid * EPT + k * CH
        pltpu.sync_copy(exb, ex_hbm.at[pl.ds(base, CH)])
        pltpu.sync_copy(exb, den_sh.at[didx.at[b]], add=True)

    stage(0, 0)

    @pl.loop(0, KCH // 2)
    def _(t):
        stage(1, 2 * t + 1)
        compute(0, 2 * t)

        @pl.when(t + 1 < KCH // 2)
        def _():
            stage(0, 2 * t + 2)

        compute(1, 2 * t + 1)

    compute_last = KCH % 2
    if compute_last:
        stage(0, KCH - 1)
        compute(0, KCH - 1)

    plsc.subcore_barrier()
    pltpu.sync_copy(den_sh.at[pl.ds(sid * NPS, NPS)],
                    den_hbm.at[cid, pl.ds(sid * NPS, NPS)])


@functools.partial(
    pl.kernel,
    out_type=_f32(NC, NP, HH, 16),
    mesh=_mesh,
    compiler_params=pltpu.CompilerParams(use_tc_tiling_on_sc=False),
    scratch_types=[
        pltpu.VMEM((2, CH), jnp.int32),
        pltpu.VMEM((2, CH), jnp.int32),
        pltpu.VMEM((2, CH, HH, 16), jnp.float32),
        pltpu.VMEM((2, CH, HH, 16), jnp.float32),
        pltpu.VMEM((2, CH, 16), jnp.float32),
        pltpu.VMEM((NPS, HH, 16), jnp.float32),
        pltpu.VMEM_SHARED((NP, HH, 16), jnp.float32),
        pltpu.SemaphoreType.DMA,
        pltpu.SemaphoreType.DMA,
        pltpu.SemaphoreType.DMA,
        pltpu.SemaphoreType.DMA,
    ],
)
def _sc_message_pass(src_hbm, dst_hbm, ex_hbm, xhh_hbm, out_hbm,
                     sidx, didx, rows, msg, exb, zb, out_sh,
                     gsem0, gsem1, ssem0, ssem1):
    cid = lax.axis_index("c")
    sid = lax.axis_index("s")
    z16 = jnp.zeros((16,), jnp.float32)
    gsems = (gsem0, gsem1)
    ssems = (ssem0, ssem1)

    @pl.loop(0, NPS)
    def _(i):
        for h in range(HH):
            zb[i, h] = z16

    pltpu.sync_copy(zb, out_sh.at[pl.ds(sid * NPS, NPS)])
    plsc.subcore_barrier()

    def wait_scatter(b):
        pltpu.make_async_copy(msg.at[b], out_sh.at[didx.at[b]],
                              ssems[b]).wait()

    def stage(b, k):
        base = sid * EPTB + k * CH
        pltpu.sync_copy(src_hbm.at[pl.ds(base, CH)], sidx.at[b])
        pltpu.sync_copy(dst_hbm.at[pl.ds(base, CH)], didx.at[b])
        pltpu.async_copy(ex_hbm.at[pl.ds(base, CH)], exb.at[b], gsems[b])
        pltpu.async_copy(xhh_hbm.at[cid].at[sidx.at[b]],
                         rows.at[b], gsems[b])

    def compute(b):
        pltpu.make_async_copy(ex_hbm.at[pl.ds(0, CH)], exb.at[b],
                              gsems[b]).wait()
        pltpu.make_async_copy(xhh_hbm.at[cid].at[sidx.at[b]],
                              rows.at[b], gsems[b]).wait()
        hb = cid * HH

        @pl.loop(0, CH)
        def _(i):
            er = exb[b, i]
            for h in range(HH):
                msg[b, i, h] = rows[b, i, h] * _splat(er, hb + h)

        pltpu.async_copy(msg.at[b], out_sh.at[didx.at[b]], ssems[b],
                         add=True)

    stage(0, 0)

    @pl.loop(0, KCHB // 2)
    def _(t):
        @pl.when(t > 0)
        def _():
            wait_scatter(1)

        stage(1, 2 * t + 1)
        compute(0)

        @pl.when(t + 1 < KCHB // 2)
        def _():
            wait_scatter(0)
            stage(0, 2 * t + 2)

        compute(1)

    wait_scatter(0)
    wait_scatter(1)
    plsc.subcore_barrier()
    pltpu.sync_copy(out_sh.at[pl.ds(sid * NPS, NPS)],
                    out_hbm.at[cid, pl.ds(sid * NPS, NPS)])


def _tc_first_body(x_ref, w0_ref, w1_ref, wa_ref, h_ref, xh_ref, al_ref):
    h = jnp.dot(x_ref[...], w0_ref[...], preferred_element_type=jnp.float32)
    h_ref[...] = h
    xh_ref[...] = jnp.dot(h, w1_ref[...], preferred_element_type=jnp.float32)
    al_ref[...] = jnp.dot(h, wa_ref[...], preferred_element_type=jnp.float32)


def _tc_first(x, w0, w1, wa):
    return pl.pallas_call(
        _tc_first_body,
        grid=(N // BM,),
        in_specs=[pl.BlockSpec((BM, D), lambda i: (i, 0)),
                  pl.BlockSpec((D, EMB), lambda i: (0, 0)),
                  pl.BlockSpec((EMB, EMB), lambda i: (0, 0)),
                  pl.BlockSpec((EMB, EMB), lambda i: (0, 0))],
        out_specs=[pl.BlockSpec((BM, EMB), lambda i: (i, 0))] * 3,
        out_shape=[_f32(N, EMB)] * 3,
    )(x, w0, w1, wa)


def _tc_post_body(h_ref, a_ref, d0_ref, d1_ref, b_ref, u_ref, st_ref):
    jrow = lax.broadcasted_iota(jnp.int32, (16, EMB), 0)
    ccol = lax.broadcasted_iota(jnp.int32, (16, EMB), 1) // C
    expm = jnp.where(jrow == ccol, 1.0, 0.0).astype(jnp.float32)
    den16 = d0_ref[0] + d1_ref[0]
    den128 = jnp.dot(den16, expm, preferred_element_type=jnp.float32)
    u = (h_ref[...]
         + a_ref[...] / (den128 + 1e-16)
         + b_ref[...])
    u_ref[...] = u

    @pl.when(pl.program_id(0) == 0)
    def _():
        st_ref[...] = jnp.zeros_like(st_ref)

    s0 = jnp.sum(u, axis=0, keepdims=True)
    s1 = jnp.sum(u * u, axis=0, keepdims=True)
    st_ref[...] += jnp.concatenate(
        [s0, s1, jnp.zeros((6, EMB), jnp.float32)], axis=0)


def _tc_post(h, agg, den, bias):
    return pl.pallas_call(
        _tc_post_body,
        grid=(N // BM,),
        in_specs=[pl.BlockSpec((BM, EMB), lambda i: (i, 0)),
                  pl.BlockSpec((BM, EMB), lambda i: (i, 0)),
                  pl.BlockSpec((1, BM, 16), lambda i: (0, i, 0)),
                  pl.BlockSpec((1, BM, 16), lambda i: (1, i, 0)),
                  pl.BlockSpec((1, EMB), lambda i: (0, 0))],
        out_specs=[pl.BlockSpec((BM, EMB), lambda i: (i, 0)),
                   pl.BlockSpec((8, EMB), lambda i: (0, 0))],
        out_shape=[_f32(N, EMB), _f32(8, EMB)],
    )(h, agg, den, den, bias)


def _tc_next_body(u_ref, sc_ref, sh_ref, w_ref, wa_ref, h_ref, xh_ref, al_ref):
    hcur = u_ref[...] * sc_ref[...] + sh_ref[...]
    h_ref[...] = hcur
    xh_ref[...] = jnp.dot(hcur, w_ref[...], preferred_element_type=jnp.float32)
    al_ref[...] = jnp.dot(hcur, wa_ref[...], preferred_element_type=jnp.float32)


def _tc_next(u, scale, shift, w, wa):
    return pl.pallas_call(
        _tc_next_body,
        grid=(N // BM,),
        in_specs=[pl.BlockSpec((BM, EMB), lambda i: (i, 0)),
                  pl.BlockSpec((1, EMB), lambda i: (0, 0)),
                  pl.BlockSpec((1, EMB), lambda i: (0, 0)),
                  pl.BlockSpec((EMB, EMB), lambda i: (0, 0)),
                  pl.BlockSpec((EMB, EMB), lambda i: (0, 0))],
        out_specs=[pl.BlockSpec((BM, EMB), lambda i: (i, 0))] * 3,
        out_shape=[_f32(N, EMB)] * 3,
    )(u, scale, shift, w, wa)


def _tc_final_body(u_ref, sc_ref, sh_ref, h_ref):
    h_ref[...] = u_ref[...] * sc_ref[...] + sh_ref[...]


def _tc_final(u, scale, shift):
    return pl.pallas_call(
        _tc_final_body,
        grid=(N // BM,),
        in_specs=[pl.BlockSpec((BM, EMB), lambda i: (i, 0)),
                  pl.BlockSpec((1, EMB), lambda i: (0, 0)),
                  pl.BlockSpec((1, EMB), lambda i: (0, 0))],
        out_specs=pl.BlockSpec((BM, EMB), lambda i: (i, 0)),
        out_shape=_f32(N, EMB),
    )(u, scale, shift)


def _prep_edges(edge_index):
    src0 = edge_index[0]
    dst0 = edge_index[1]
    loops = jnp.arange(N, dtype=jnp.int32)
    keep = src0 != dst0
    padn = E_PAD - (E + N)
    src = jnp.concatenate([src0, loops, jnp.zeros((padn,), jnp.int32)])
    dst = jnp.concatenate([jnp.where(keep, dst0, N), loops,
                           jnp.full((padn,), N, jnp.int32)])
    return src, dst


def _attn_w(w, a_s, a_d):
    wr = w.reshape(EMB, H, C)
    ws = (wr * a_s[None]).sum(-1)
    wd = (wr * a_d[None]).sum(-1)
    wa = jnp.concatenate([ws, wd], axis=1)
    return jnp.pad(wa, ((0, 0), (0, EMB - 2 * H)))


def _al_tables(al):
    alsd = al[:, :16]
    alds = jnp.concatenate([al[:, 8:16], al[:, :8]], axis=1)
    pad = ((0, NP - N), (0, 0))
    return jnp.pad(alsd, pad), jnp.pad(alds, pad)


def _bn_coeffs(st, g, bt):
    mean = st[0] / N
    var = st[1] / N - mean * mean
    s = g * lax.rsqrt(var + 1e-5)
    return s.reshape(1, EMB), (bt - mean * s).reshape(1, EMB)


def _gat_residual(h, xh, al, src, dst, bias):
    alsd, alds = _al_tables(al)
    ex, den = _sc_edge_denom(src, dst, alsd, alds)
    xhh = xh.reshape(N, NC, HH, 16).transpose(1, 0, 2, 3)
    outp = _sc_message_pass(src, dst, ex, xhh)
    agg = jnp.concatenate([outp[0, :N], outp[1, :N]], axis=1).reshape(N, EMB)
    return _tc_post(h, agg, den, bias.reshape(1, EMB))


def kernel(x, edge_index, W0, W1, as1, ad1, b1, g1, bt1,
           W2, as2, ad2, b2, g2, bt2, W3, as3, ad3, b3, g3, bt3):
    src, dst = _prep_edges(edge_index)
    h, xh, al = _tc_first(x, W0, W1, _attn_w(W1, as1, ad1))
    u, st = _gat_residual(h, xh, al, src, dst, b1)
    sc_, sh_ = _bn_coeffs(st, g1, bt1)
    h, xh, al = _tc_next(u, sc_, sh_, W2, _attn_w(W2, as2, ad2))
    u, st = _gat_residual(h, xh, al, src, dst, b2)
    sc_, sh_ = _bn_coeffs(st, g2, bt2)
    h, xh, al = _tc_next(u, sc_, sh_, W3, _attn_w(W3, as3, ad3))
    u, st = _gat_residual(h, xh, al, src, dst, b3)
    sc_, sh_ = _bn_coeffs(st, g3, bt3)
    return _tc_final(u, sc_, sh_)
